# Initial kernel scaffold; baseline (speedup 1.0000x reference)
import functools

import jax
import jax.numpy as jnp
from jax import lax
from jax.experimental import pallas as pl
from jax.experimental.pallas import tpu as pltpu
from jax.experimental.pallas import tpu_sc as plsc

N = 10000
E = 320000
D = 128
H = 512
ENC = 256
PLH = 64
B = 100000
MLP = 25

NC, NS = 2, 16
ET = E // NS


DEG_ROWS = 100
DEG_PAD = 10240


def _sc_degree_body(dst_hbm, out_hbm, dst_v, ones_v, zbuf, acc):
    c = lax.axis_index("c")
    s = lax.axis_index("s")
    w = c * NS + s

    def fill_z(k, _):
        zbuf[pl.ds(k * 16, 16)] = jnp.zeros((16,), jnp.float32)
        return 0
    lax.fori_loop(0, 40, fill_z, 0)
    for off in (0, 16, 32, 48, 64, 80, 84):
        ones_v[pl.ds(off, 16)] = jnp.ones((16,), jnp.float32)

    pltpu.sync_copy(zbuf, acc.at[pl.ds(s * 640, 640)])
    plsc.subcore_barrier()

    pltpu.sync_copy(dst_hbm.at[pl.ds(w * DEG_ROWS, DEG_ROWS), :], dst_v)

    def row(j, _):
        pltpu.sync_copy(ones_v, acc.at[dst_v.at[j]], add=True)
        return 0
    lax.fori_loop(0, DEG_ROWS, row, 0)
    plsc.subcore_barrier()

    pltpu.sync_copy(acc.at[pl.ds(s * 640, 640)], out_hbm.at[c, pl.ds(s * 640, 640)])


_sc_degree = functools.partial(
    pl.kernel,
    out_type=jax.ShapeDtypeStruct((NC, DEG_PAD), jnp.float32),
    mesh=plsc.VectorSubcoreMesh(core_axis_name="c", subcore_axis_name="s"),
    scratch_types=[
        pltpu.VMEM((E // DEG_ROWS // (NC * NS), DEG_ROWS), jnp.int32),
        pltpu.VMEM((100,), jnp.float32),
        pltpu.VMEM((640,), jnp.float32),
        pltpu.VMEM_SHARED((DEG_PAD,), jnp.float32),
    ],
)(_sc_degree_body)


RSZ = 2512
ACC_R = 2560
TRASH = 2512
CH = 48
ECH = 4000
SEL = ET + 96


def _sc_msgpass_body(F, src_hbm, dst_hbm, tab_hbm, out_hbm,
                     es, ed, sel_s, sel_d, zbuf, rows, idxw, acc, sem):
    c = lax.axis_index("c")
    s = lax.axis_index("s")

    def fill_z(k, _):
        r = k // (F // 16)
        col = k % (F // 16)
        zbuf[r, pl.ds(col * 16, 16)] = jnp.zeros((16,), jnp.float32)
        return 0
    lax.fori_loop(0, 16 * (F // 16), fill_z, 0)

    for rr in range(2):
        rng = c * 2 + rr
        base = rng * RSZ
        limit = jnp.where(rng == 3, N - 3 * RSZ, RSZ)

        def zero_k(k, _):
            pltpu.sync_copy(zbuf, acc.at[pl.ds(s * 160 + k * 16, 16), :])
            return 0
        lax.fori_loop(0, 10, zero_k, 0)
        plsc.subcore_barrier()

        def scan_chunk(ec, cnt):
            off = s * ET + ec * ECH
            pltpu.sync_copy(src_hbm.at[pl.ds(off, ECH)], es)
            pltpu.sync_copy(dst_hbm.at[pl.ds(off, ECH)], ed)

            def scan_vec(i, cnt):
                sv = es[pl.ds(i * 16, 16)]
                dv = ed[pl.ds(i * 16, 16)]
                m = (dv >= base) & (dv < base + RSZ)
                plsc.store_compressed(sel_s.at[pl.ds(cnt, 16)], sv, mask=m)
                plsc.store_compressed(sel_d.at[pl.ds(cnt, 16)], dv - base,
                                      mask=m)
                return cnt + jnp.sum(m.astype(jnp.int32))
            return lax.fori_loop(0, ECH // 16, scan_vec, cnt)
        cnt = lax.fori_loop(0, ET // ECH, scan_chunk, jnp.int32(0))

        for t in range(CH // 16):
            sel_s[pl.ds(cnt + t * 16, 16)] = jnp.zeros((16,), jnp.int32)
            sel_d[pl.ds(cnt + t * 16, 16)] = jnp.full((16,), TRASH, jnp.int32)
        nch = (cnt + (CH - 1)) // CH

        @pl.when(nch > 0)
        def _():
            pltpu.async_copy(tab_hbm.at[sel_s.at[pl.ds(0, CH)]], rows.at[0],
                             sem)

        def chunk(j, _):
            r = jnp.bitwise_and(j, 1)
            pltpu.make_async_copy(tab_hbm.at[pl.ds(0, CH)], rows.at[r],
                                  sem).wait()

            @pl.when(j + 1 < nch)
            def _():
                pltpu.async_copy(tab_hbm.at[sel_s.at[pl.ds((j + 1) * CH, CH)]],
                                 rows.at[1 - r], sem)

            def cp_idx(t, _):
                idxw[r, pl.ds(t * 16, 16)] = sel_d[pl.ds(j * CH + t * 16, 16)]
                return 0
            lax.fori_loop(0, CH // 16, cp_idx, 0)
            pltpu.sync_copy(rows.at[r], acc.at[idxw.at[r]], add=True)
            return 0
        lax.fori_loop(0, nch, chunk, 0)
        plsc.subcore_barrier()

        def wb(k, _):
            row0 = s * 160 + k * 16

            @pl.when(row0 < limit)
            def _():
                pltpu.sync_copy(acc.at[pl.ds(row0, 16), :],
                                out_hbm.at[pl.ds(base + row0, 16), :])
            return 0
        lax.fori_loop(0, 10, wb, 0)
        if rr == 0:
            plsc.subcore_barrier()


def _make_msgpass(F):
    return functools.partial(
        pl.kernel,
        out_type=jax.ShapeDtypeStruct((N, F), jnp.float32),
        mesh=plsc.VectorSubcoreMesh(core_axis_name="c", subcore_axis_name="s"),
        scratch_types=[
            pltpu.VMEM((ECH,), jnp.int32),
            pltpu.VMEM((ECH,), jnp.int32),
            pltpu.VMEM((SEL,), jnp.int32),
            pltpu.VMEM((SEL,), jnp.int32),
            pltpu.VMEM((16, F), jnp.float32),
            pltpu.VMEM((2, CH, F), jnp.float32),
            pltpu.VMEM((2, CH), jnp.int32),
            pltpu.VMEM_SHARED((ACC_R, F), jnp.float32),
            pltpu.SemaphoreType.DMA,
        ],
    )(functools.partial(_sc_msgpass_body, F))


_sc_msgpass_h = _make_msgpass(H)
_sc_msgpass_e = _make_msgpass(ENC)


BPAD = 100352
QT = BPAD // 32
C2 = 56
NCH2 = QT // C2


def _sc_sqdist_body(ein_hbm, eout_hbm, emb_hbm, out_hbm,
                    ei, eo, ri, ro, sem):
    c = lax.axis_index("c")
    s = lax.axis_index("s")
    w = c * NS + s
    qbase = w * QT

    pltpu.sync_copy(ein_hbm.at[pl.ds(qbase, QT)], ei)
    pltpu.sync_copy(eout_hbm.at[pl.ds(qbase, QT)], eo)

    pltpu.async_copy(emb_hbm.at[ei.at[pl.ds(0, C2)]], ri.at[0], sem)
    pltpu.async_copy(emb_hbm.at[eo.at[pl.ds(0, C2)]], ro.at[0], sem)

    def chunk(j, _):
        r = jnp.bitwise_and(j, 1)
        pltpu.make_async_copy(emb_hbm.at[pl.ds(0, C2)], ri.at[r], sem).wait()
        pltpu.make_async_copy(emb_hbm.at[pl.ds(0, C2)], ro.at[r], sem).wait()

        @pl.when(j + 1 < NCH2)
        def _():
            pltpu.async_copy(emb_hbm.at[ei.at[pl.ds((j + 1) * C2, C2)]],
                             ri.at[1 - r], sem)
            pltpu.async_copy(emb_hbm.at[eo.at[pl.ds((j + 1) * C2, C2)]],
                             ro.at[1 - r], sem)

        def rowfn(a, _):
            for bcol in range(ENC // 16):
                x = ri[r, a, pl.ds(bcol * 16, 16)]
                y = ro[r, a, pl.ds(bcol * 16, 16)]
                d = x - y
                ri[r, a, pl.ds(bcol * 16, 16)] = d * d
            return 0
        lax.fori_loop(0, C2, rowfn, 0)
        pltpu.sync_copy(ri.at[r], out_hbm.at[pl.ds(qbase + j * C2, C2), :])
        return 0
    lax.fori_loop(0, NCH2, chunk, 0)


_sc_sqdist = functools.partial(
    pl.kernel,
    out_type=jax.ShapeDtypeStruct((BPAD, ENC), jnp.float32),
    mesh=plsc.VectorSubcoreMesh(core_axis_name="c", subcore_axis_name="s"),
    scratch_types=[
        pltpu.VMEM((QT,), jnp.int32),
        pltpu.VMEM((QT,), jnp.int32),
        pltpu.VMEM((2, C2, ENC), jnp.float32),
        pltpu.VMEM((2, C2, ENC), jnp.float32),
        pltpu.SemaphoreType.DMA,
    ],
)(_sc_sqdist_body)


BM = 1000


def _tc_enc1_body(x_ref, w1_ref, d0_ref, d1_ref, xt1_ref, dinv_ref):
    dv = lax.rsqrt(d0_ref[...] + d1_ref[...] + 1.0)
    xw = jnp.dot(x_ref[...], w1_ref[...], preferred_element_type=jnp.float32)
    xt1_ref[...] = xw * dv
    dinv_ref[...] = dv


def _tc_enc1(x, w1, d0, d1):
    return pl.pallas_call(
        _tc_enc1_body,
        grid=(N // BM,),
        in_specs=[
            pl.BlockSpec((BM, D), lambda i: (i, 0)),
            pl.BlockSpec((D, H), lambda i: (0, 0)),
            pl.BlockSpec((BM, 1), lambda i: (i, 0)),
            pl.BlockSpec((BM, 1), lambda i: (i, 0)),
        ],
        out_specs=[
            pl.BlockSpec((BM, H), lambda i: (i, 0)),
            pl.BlockSpec((BM, 1), lambda i: (i, 0)),
        ],
        out_shape=[
            jax.ShapeDtypeStruct((N, H), jnp.float32),
            jax.ShapeDtypeStruct((N, 1), jnp.float32),
        ],
    )(x, w1, d0, d1)


def _tc_enc2_body(s1_ref, xt1_ref, dinv_ref, b1_ref, w2_ref, xt2_ref):
    h = (s1_ref[...] + xt1_ref[...]) * dinv_ref[...] + b1_ref[...]
    h = jnp.maximum(h, 0.0)
    xw = jnp.dot(h, w2_ref[...], preferred_element_type=jnp.float32)
    xt2_ref[...] = xw * dinv_ref[...]


def _tc_enc2(s1, xt1, dinv, b1, w2):
    return pl.pallas_call(
        _tc_enc2_body,
        grid=(N // BM,),
        in_specs=[
            pl.BlockSpec((BM, H), lambda i: (i, 0)),
            pl.BlockSpec((BM, H), lambda i: (i, 0)),
            pl.BlockSpec((BM, 1), lambda i: (i, 0)),
            pl.BlockSpec((1, H), lambda i: (0, 0)),
            pl.BlockSpec((H, ENC), lambda i: (0, 0)),
        ],
        out_specs=pl.BlockSpec((BM, ENC), lambda i: (i, 0)),
        out_shape=jax.ShapeDtypeStruct((N, ENC), jnp.float32),
    )(s1, xt1, dinv, b1, w2)


def _tc_emb_body(s2_ref, xt2_ref, dinv_ref, b2_ref, emb_ref):
    z = (s2_ref[...] + xt2_ref[...]) * dinv_ref[...] + b2_ref[...]
    emb_ref[...] = jnp.maximum(z, 0.0)


def _tc_emb(s2, xt2, dinv, b2):
    return pl.pallas_call(
        _tc_emb_body,
        grid=(N // BM,),
        in_specs=[
            pl.BlockSpec((BM, ENC), lambda i: (i, 0)),
            pl.BlockSpec((BM, ENC), lambda i: (i, 0)),
            pl.BlockSpec((BM, 1), lambda i: (i, 0)),
            pl.BlockSpec((1, ENC), lambda i: (0, 0)),
        ],
        out_specs=pl.BlockSpec((BM, ENC), lambda i: (i, 0)),
        out_shape=jax.ShapeDtypeStruct((N, ENC), jnp.float32),
    )(s2, xt2, dinv, b2)


BD = 2000


def _tc_dec_body(sq_ref, plh_ref, wa_ref, wb_ref, db1_ref, dw2_ref, db2_ref,
                 out_ref):
    z = (jnp.dot(sq_ref[...], wa_ref[...], preferred_element_type=jnp.float32)
         + jnp.dot(plh_ref[...], wb_ref[...],
                   preferred_element_type=jnp.float32)
         + db1_ref[...])
    z = jnp.where(z >= 0.0, z, 0.1 * z)
    z = jnp.dot(z, dw2_ref[...], preferred_element_type=jnp.float32) \
        + db2_ref[...]
    z = jnp.clip(jnp.abs(z), 0.0, 40.0)
    out_ref[...] = 1.0 / (jnp.exp((z - 2.0) * 2.0) + 1.0)


def _tc_dec(sq, plh, wa, wb, db1, dw2, db2):
    return pl.pallas_call(
        _tc_dec_body,
        grid=(B // BD,),
        in_specs=[
            pl.BlockSpec((BD, ENC), lambda i: (i, 0)),
            pl.BlockSpec((BD, PLH), lambda i: (i, 0)),
            pl.BlockSpec((ENC, MLP), lambda i: (0, 0)),
            pl.BlockSpec((PLH, MLP), lambda i: (0, 0)),
            pl.BlockSpec((1, MLP), lambda i: (0, 0)),
            pl.BlockSpec((MLP, 1), lambda i: (0, 0)),
            pl.BlockSpec((1, 1), lambda i: (0, 0)),
        ],
        out_specs=pl.BlockSpec((BD, 1), lambda i: (i, 0)),
        out_shape=jax.ShapeDtypeStruct((B, 1), jnp.float32),
    )(sq, plh, wa, wb, db1, dw2, db2)


def kernel(node_x, gnn_edge_index, edges, plh_x, W1, b1, W2, b2,
           dW1, db1, dW2, db2):
    src = gnn_edge_index[0]
    dst = gnn_edge_index[1]

    degp = _sc_degree(dst.reshape(E // DEG_ROWS, DEG_ROWS))
    d0 = degp[0, :N].reshape(N, 1)
    d1 = degp[1, :N].reshape(N, 1)

    xt1, dinv = _tc_enc1(node_x, W1, d0, d1)
    s1 = _sc_msgpass_h(src, dst, xt1)
    xt2 = _tc_enc2(s1, xt1, dinv, b1.reshape(1, H), W2)
    s2 = _sc_msgpass_e(src, dst, xt2)
    emb = _tc_emb(s2, xt2, dinv, b2.reshape(1, ENC))

    ein = jnp.pad(edges[:, 0], (0, BPAD - B))
    eout = jnp.pad(edges[:, 1], (0, BPAD - B))
    sq = _sc_sqdist(ein, eout, emb)

    out = _tc_dec(sq, plh_x, dW1[:ENC], dW1[ENC:], db1.reshape(1, MLP),
                  dW2, db2.reshape(1, 1))
    return out.reshape(-1)


# speedup vs baseline: 3.5461x; 3.5461x over previous
import functools

import jax
import jax.numpy as jnp
from jax import lax
from jax.experimental import pallas as pl
from jax.experimental.pallas import tpu as pltpu
from jax.experimental.pallas import tpu_sc as plsc

N = 10000
E = 320000
D = 128
H = 512
ENC = 256
PLH = 64
B = 100000
MLP = 25

NC, NS = 2, 16
ECH = 4096
EPT = 6 * ECH
EPAD = NS * EPT
DST_PAD = 16384


DEG_W = 128
DEG_RPT = 80
DEG_PAD = 10240


def _sc_degree_body(dst_hbm, out_hbm, dst_v, ones_v, zbuf, acc):
    c = lax.axis_index("c")
    s = lax.axis_index("s")
    w = c * NS + s

    def fill_z(k, _):
        zbuf[pl.ds(k * 16, 16)] = jnp.zeros((16,), jnp.float32)
        return 0
    lax.fori_loop(0, 40, fill_z, 0)
    for off in range(0, DEG_W, 16):
        ones_v[pl.ds(off, 16)] = jnp.ones((16,), jnp.float32)

    pltpu.sync_copy(zbuf, acc.at[pl.ds(s * 640, 640)])
    plsc.subcore_barrier()

    pltpu.sync_copy(dst_hbm.at[pl.ds(w * DEG_RPT, DEG_RPT), :], dst_v)

    def row(j, _):
        pltpu.sync_copy(ones_v, acc.at[dst_v.at[j]], add=True)
        return 0
    lax.fori_loop(0, DEG_RPT, row, 0)
    plsc.subcore_barrier()

    pltpu.sync_copy(acc.at[pl.ds(s * 640, 640)], out_hbm.at[c, pl.ds(s * 640, 640)])


_sc_degree = functools.partial(
    pl.kernel,
    out_type=jax.ShapeDtypeStruct((NC, DEG_PAD), jnp.float32),
    mesh=plsc.VectorSubcoreMesh(core_axis_name="c", subcore_axis_name="s", num_cores=NC, num_subcores=NS),
    compiler_params=pltpu.CompilerParams(use_tc_tiling_on_sc=False, needs_layout_passes=False),
    scratch_types=[
        pltpu.VMEM((DEG_RPT, DEG_W), jnp.int32),
        pltpu.VMEM((DEG_W,), jnp.float32),
        pltpu.VMEM((640,), jnp.float32),
        pltpu.VMEM_SHARED((DEG_PAD,), jnp.float32),
    ],
)(_sc_degree_body)


NRANGE = 8
RSZ = 1280
ACC_R = 1344
TRASH = 1280
PACK_BITS = 11
PACK = 1 << PACK_BITS
CH = 48
NSUB = EPT // ECH
LCAP = 4160
MAXCH = (ECH + CH - 1) // CH


def _sc_compact_body(src_hbm, dst_hbm, codes_hbm, es, ed, sel):
    c = lax.axis_index("c")
    s = lax.axis_index("s")
    w = c * NS + s

    def edge_chunk(ec, _):
        off = s * EPT + ec * ECH
        pltpu.sync_copy(src_hbm.at[pl.ds(off, ECH)], es)
        pltpu.sync_copy(dst_hbm.at[pl.ds(off, ECH)], ed)

        for rng in range(NRANGE):
            base = rng * RSZ

            def scan_vec(i, cnt):
                sv = es[pl.ds(i * 16, 16)]
                dv = ed[pl.ds(i * 16, 16)]
                m = (dv >= base) & (dv < base + RSZ)
                mi = m.astype(jnp.int32)
                pos = 16 + cnt + plsc.cumsum(mi) - 1
                code = sv * PACK + (dv - base)
                plsc.store_scatter(sel, [pos], code, mask=m)
                return cnt + jnp.sum(mi)
            cnt = lax.fori_loop(0, ECH // 16, scan_vec, jnp.int32(0))

            sel[pl.ds(0, 16)] = jnp.full((16,), 1, jnp.int32) * cnt
            for t in range(CH // 16):
                sel[pl.ds(16 + cnt + t * 16, 16)] = jnp.full(
                    (16,), TRASH, jnp.int32)
            pltpu.sync_copy(sel, codes_hbm.at[rng, w, ec])
        return 0
    lax.fori_loop(0, NSUB, edge_chunk, 0)


_sc_compact = functools.partial(
    pl.kernel,
    out_type=jax.ShapeDtypeStruct((NRANGE, NC * NS, NSUB, LCAP), jnp.int32),
    mesh=plsc.VectorSubcoreMesh(core_axis_name="c", subcore_axis_name="s", num_cores=NC, num_subcores=NS),
    compiler_params=pltpu.CompilerParams(use_tc_tiling_on_sc=False, needs_layout_passes=False),
    scratch_types=[
        pltpu.VMEM((ECH,), jnp.int32),
        pltpu.VMEM((ECH,), jnp.int32),
        pltpu.VMEM((LCAP,), jnp.int32),
    ],
)(_sc_compact_body)


def _sc_scatter_body(F, codes_hbm, tab_hbm, out_hbm,
                     sel, zbuf, rows, idxw, acc, sem):
    c = lax.axis_index("c")
    s = lax.axis_index("s")

    for zr in range(16):
        def fill_z(col, _, zr=zr):
            zbuf[zr, pl.ds(col * 16, 16)] = jnp.zeros((16,), jnp.float32)
            return 0
        lax.fori_loop(0, F // 16, fill_z, 0)

    def decode(j, r):
        def dec(t, _):
            cv = sel[pl.ds(16 + j * CH + t * 16, 16)]
            idxw[2 + r, pl.ds(t * 16, 16)] = lax.shift_right_logical(
                cv, PACK_BITS)
            idxw[r, pl.ds(t * 16, 16)] = jnp.bitwise_and(cv, PACK - 1)
            return 0
        lax.fori_loop(0, CH // 16, dec, 0)

    for rr in range(NRANGE // NC):
        rng_s0 = rr
        rng_s1 = 4 + rr
        base = (c * (NRANGE // NC) + rr) * RSZ
        limit = jnp.where(c * (NRANGE // NC) + rr == NRANGE - 1,
                          N - (NRANGE - 1) * RSZ, RSZ)

        def zero_k(k, _):
            @pl.when(k % NS == s)
            def _():
                pltpu.sync_copy(zbuf, acc.at[pl.ds(k * 16, 16), :])
            return 0
        lax.fori_loop(0, ACC_R // 16, zero_k, 0)
        plsc.subcore_barrier()

        def sublist(t2, _):
            plist = 2 * s + (t2 // NSUB)
            sub = t2 % NSUB

            @pl.when(c == 0)
            def _():
                pltpu.sync_copy(codes_hbm.at[rng_s0, plist, sub], sel)

            @pl.when(c == 1)
            def _():
                pltpu.sync_copy(codes_hbm.at[rng_s1, plist, sub], sel)

            cnt = lax.shift_right_logical(jnp.sum(sel[pl.ds(0, 16)]), 4)
            nch = (cnt + (CH - 1)) // CH

            @pl.when(nch > 0)
            def _():
                decode(jnp.int32(0), 0)
                pltpu.async_copy(tab_hbm.at[idxw.at[2]], rows.at[0], sem)

            def chunk2(jj, _):
                for r in (0, 1):
                    j = 2 * jj + r

                    @pl.when(j < nch)
                    def _(j=j, r=r):
                        pltpu.make_async_copy(tab_hbm.at[pl.ds(0, CH)],
                                              rows.at[r], sem).wait()

                        @pl.when(j + 1 < nch)
                        def _(j=j, r=r):
                            decode(j + 1, 1 - r)
                            pltpu.async_copy(tab_hbm.at[idxw.at[2 + (1 - r)]],
                                             rows.at[1 - r], sem)
                        pltpu.sync_copy(rows.at[r], acc.at[idxw.at[r]],
                                        add=True)
                return 0
            lax.fori_loop(0, (MAXCH + 1) // 2, chunk2, 0)
            return 0
        lax.fori_loop(0, 2 * NSUB, sublist, 0)
        plsc.subcore_barrier()

        def wb(k, _):
            @pl.when((k % NS == s) & (k * 16 < limit))
            def _():
                pltpu.sync_copy(acc.at[pl.ds(k * 16, 16), :],
                                out_hbm.at[pl.ds(base + k * 16, 16), :])
            return 0
        lax.fori_loop(0, RSZ // 16, wb, 0)
        if rr != NRANGE // NC - 1:
            plsc.subcore_barrier()


def _make_scatter(F):
    return functools.partial(
        pl.kernel,
        out_type=jax.ShapeDtypeStruct((N, F), jnp.float32),
        mesh=plsc.VectorSubcoreMesh(core_axis_name="c", subcore_axis_name="s", num_cores=NC, num_subcores=NS),
        compiler_params=pltpu.CompilerParams(use_tc_tiling_on_sc=False, needs_layout_passes=False),
        scratch_types=[
            pltpu.VMEM((LCAP,), jnp.int32),
            pltpu.VMEM((16, F), jnp.float32),
            pltpu.VMEM((2, CH, F), jnp.float32),
            pltpu.VMEM((4, CH), jnp.int32),
            pltpu.VMEM_SHARED((ACC_R, F), jnp.float32),
            pltpu.SemaphoreType.DMA,
        ],
    )(functools.partial(_sc_scatter_body, F))


_sc_scatter_h = _make_scatter(H)
_sc_scatter_e = _make_scatter(ENC)


BPAD = 100352
QT = BPAD // 32
C2 = 56
NCH2 = QT // C2


def _sc_sqdist_body(ein_hbm, eout_hbm, emb_hbm, out_hbm,
                    ei, eo, ri, ro, sem):
    c = lax.axis_index("c")
    s = lax.axis_index("s")
    w = c * NS + s
    qbase = w * QT

    pltpu.sync_copy(ein_hbm.at[pl.ds(qbase, QT)], ei)
    pltpu.sync_copy(eout_hbm.at[pl.ds(qbase, QT)], eo)

    pltpu.async_copy(emb_hbm.at[ei.at[pl.ds(0, C2)]], ri.at[0], sem)
    pltpu.async_copy(emb_hbm.at[eo.at[pl.ds(0, C2)]], ro.at[0], sem)

    def chunk(j, _):
        r = jnp.bitwise_and(j, 1)
        pltpu.make_async_copy(emb_hbm.at[pl.ds(0, C2)], ri.at[r], sem).wait()
        pltpu.make_async_copy(emb_hbm.at[pl.ds(0, C2)], ro.at[r], sem).wait()

        @pl.when(j + 1 < NCH2)
        def _():
            pltpu.async_copy(emb_hbm.at[ei.at[pl.ds((j + 1) * C2, C2)]],
                             ri.at[1 - r], sem)
            pltpu.async_copy(emb_hbm.at[eo.at[pl.ds((j + 1) * C2, C2)]],
                             ro.at[1 - r], sem)

        def rowfn(a, _):
            for bcol in range(ENC // 16):
                x = ri[r, a, pl.ds(bcol * 16, 16)]
                y = ro[r, a, pl.ds(bcol * 16, 16)]
                d = x - y
                ri[r, a, pl.ds(bcol * 16, 16)] = d * d
            return 0
        lax.fori_loop(0, C2, rowfn, 0)
        pltpu.sync_copy(ri.at[r], out_hbm.at[pl.ds(qbase + j * C2, C2), :])
        return 0
    lax.fori_loop(0, NCH2, chunk, 0)


_sc_sqdist = functools.partial(
    pl.kernel,
    out_type=jax.ShapeDtypeStruct((BPAD, ENC), jnp.float32),
    mesh=plsc.VectorSubcoreMesh(core_axis_name="c", subcore_axis_name="s", num_cores=NC, num_subcores=NS),
        compiler_params=pltpu.CompilerParams(use_tc_tiling_on_sc=False, needs_layout_passes=False),
    scratch_types=[
        pltpu.VMEM((QT,), jnp.int32),
        pltpu.VMEM((QT,), jnp.int32),
        pltpu.VMEM((2, C2, ENC), jnp.float32),
        pltpu.VMEM((2, C2, ENC), jnp.float32),
        pltpu.SemaphoreType.DMA,
    ],
)(_sc_sqdist_body)


BM = 1000


def _tc_enc1_body(x_ref, w1_ref, d0_ref, d1_ref, xt1_ref, dinv_ref):
    dv = lax.rsqrt(d0_ref[...] + d1_ref[...] + 1.0)
    xw = jnp.dot(x_ref[...], w1_ref[...], preferred_element_type=jnp.float32)
    xt1_ref[...] = xw * dv
    dinv_ref[...] = dv


def _tc_enc1(x, w1, d0, d1):
    return pl.pallas_call(
        _tc_enc1_body,
        grid=(N // BM,),
        in_specs=[
            pl.BlockSpec((BM, D), lambda i: (i, 0)),
            pl.BlockSpec((D, H), lambda i: (0, 0)),
            pl.BlockSpec((BM, 1), lambda i: (i, 0)),
            pl.BlockSpec((BM, 1), lambda i: (i, 0)),
        ],
        out_specs=[
            pl.BlockSpec((BM, H), lambda i: (i, 0)),
            pl.BlockSpec((BM, 1), lambda i: (i, 0)),
        ],
        out_shape=[
            jax.ShapeDtypeStruct((N, H), jnp.float32),
            jax.ShapeDtypeStruct((N, 1), jnp.float32),
        ],
    )(x, w1, d0, d1)


def _tc_enc2_body(s1_ref, xt1_ref, dinv_ref, b1_ref, w2_ref, xt2_ref):
    h = (s1_ref[...] + xt1_ref[...]) * dinv_ref[...] + b1_ref[...]
    h = jnp.maximum(h, 0.0)
    xw = jnp.dot(h, w2_ref[...], preferred_element_type=jnp.float32)
    xt2_ref[...] = xw * dinv_ref[...]


def _tc_enc2(s1, xt1, dinv, b1, w2):
    return pl.pallas_call(
        _tc_enc2_body,
        grid=(N // BM,),
        in_specs=[
            pl.BlockSpec((BM, H), lambda i: (i, 0)),
            pl.BlockSpec((BM, H), lambda i: (i, 0)),
            pl.BlockSpec((BM, 1), lambda i: (i, 0)),
            pl.BlockSpec((1, H), lambda i: (0, 0)),
            pl.BlockSpec((H, ENC), lambda i: (0, 0)),
        ],
        out_specs=pl.BlockSpec((BM, ENC), lambda i: (i, 0)),
        out_shape=jax.ShapeDtypeStruct((N, ENC), jnp.float32),
    )(s1, xt1, dinv, b1, w2)


def _tc_emb_body(s2_ref, xt2_ref, dinv_ref, b2_ref, emb_ref):
    z = (s2_ref[...] + xt2_ref[...]) * dinv_ref[...] + b2_ref[...]
    emb_ref[...] = jnp.maximum(z, 0.0)


def _tc_emb(s2, xt2, dinv, b2):
    return pl.pallas_call(
        _tc_emb_body,
        grid=(N // BM,),
        in_specs=[
            pl.BlockSpec((BM, ENC), lambda i: (i, 0)),
            pl.BlockSpec((BM, ENC), lambda i: (i, 0)),
            pl.BlockSpec((BM, 1), lambda i: (i, 0)),
            pl.BlockSpec((1, ENC), lambda i: (0, 0)),
        ],
        out_specs=pl.BlockSpec((BM, ENC), lambda i: (i, 0)),
        out_shape=jax.ShapeDtypeStruct((N, ENC), jnp.float32),
    )(s2, xt2, dinv, b2)


BD = 2000


def _tc_dec_body(sq_ref, plh_ref, wa_ref, wb_ref, db1_ref, dw2_ref, db2_ref,
                 out_ref):
    z = (jnp.dot(sq_ref[...], wa_ref[...], preferred_element_type=jnp.float32)
         + jnp.dot(plh_ref[...], wb_ref[...],
                   preferred_element_type=jnp.float32)
         + db1_ref[...])
    z = jnp.where(z >= 0.0, z, 0.1 * z)
    z = jnp.dot(z, dw2_ref[...], preferred_element_type=jnp.float32) \
        + db2_ref[...]
    z = jnp.clip(jnp.abs(z), 0.0, 40.0)
    out_ref[...] = 1.0 / (jnp.exp((z - 2.0) * 2.0) + 1.0)


def _tc_dec(sq, plh, wa, wb, db1, dw2, db2):
    return pl.pallas_call(
        _tc_dec_body,
        grid=(B // BD,),
        in_specs=[
            pl.BlockSpec((BD, ENC), lambda i: (i, 0)),
            pl.BlockSpec((BD, PLH), lambda i: (i, 0)),
            pl.BlockSpec((ENC, MLP), lambda i: (0, 0)),
            pl.BlockSpec((PLH, MLP), lambda i: (0, 0)),
            pl.BlockSpec((1, MLP), lambda i: (0, 0)),
            pl.BlockSpec((MLP, 1), lambda i: (0, 0)),
            pl.BlockSpec((1, 1), lambda i: (0, 0)),
        ],
        out_specs=pl.BlockSpec((BD, 1), lambda i: (i, 0)),
        out_shape=jax.ShapeDtypeStruct((B, 1), jnp.float32),
    )(sq, plh, wa, wb, db1, dw2, db2)


def kernel(node_x, gnn_edge_index, edges, plh_x, W1, b1, W2, b2,
           dW1, db1, dW2, db2):
    src = gnn_edge_index[0]
    dst = gnn_edge_index[1]
    srcp = jnp.pad(src, (0, EPAD - E))
    dstp_e = jnp.pad(dst, (0, EPAD - E), constant_values=DST_PAD)

    dstp = jnp.concatenate(
        [dst, jnp.full((NC * NS * DEG_RPT * DEG_W - E,), N, jnp.int32)]
    ).reshape(NC * NS * DEG_RPT, DEG_W)
    degp = _sc_degree(dstp)
    d0 = degp[0, :N].reshape(N, 1)
    d1 = degp[1, :N].reshape(N, 1)

    xt1, dinv = _tc_enc1(node_x, W1, d0, d1)
    codes = _sc_compact(srcp, dstp_e)
    s1 = _sc_scatter_h(codes, xt1)
    xt2 = _tc_enc2(s1, xt1, dinv, b1.reshape(1, H), W2)
    s2 = _sc_scatter_e(codes, xt2)
    emb = _tc_emb(s2, xt2, dinv, b2.reshape(1, ENC))

    ein = jnp.pad(edges[:, 0], (0, BPAD - B))
    eout = jnp.pad(edges[:, 1], (0, BPAD - B))
    sq = _sc_sqdist(ein, eout, emb)

    out = _tc_dec(sq, plh_x, dW1[:ENC], dW1[ENC:], db1.reshape(1, MLP),
                  dW2, db2.reshape(1, 1))
    return out.reshape(-1)


# speedup vs baseline: 3.7085x; 1.0458x over previous
import functools

import jax
import jax.numpy as jnp
from jax import lax
from jax.experimental import pallas as pl
from jax.experimental.pallas import tpu as pltpu
from jax.experimental.pallas import tpu_sc as plsc

N = 10000
E = 320000
D = 128
H = 512
ENC = 256
PLH = 64
B = 100000
MLP = 25

NC, NS = 2, 16
ECH = 4096
EPT = 6 * ECH
EPAD = NS * EPT
DST_PAD = 16384


DEG_W = 128
DEG_RPT = 80
DEG_PAD = 10240


def _sc_degree_body(dst_hbm, out_hbm, dst_v, ones_v, zbuf, acc):
    c = lax.axis_index("c")
    s = lax.axis_index("s")
    w = c * NS + s

    def fill_z(k, _):
        zbuf[pl.ds(k * 16, 16)] = jnp.zeros((16,), jnp.float32)
        return 0
    lax.fori_loop(0, 40, fill_z, 0)
    for off in range(0, DEG_W, 16):
        ones_v[pl.ds(off, 16)] = jnp.ones((16,), jnp.float32)

    pltpu.sync_copy(zbuf, acc.at[pl.ds(s * 640, 640)])
    plsc.subcore_barrier()

    pltpu.sync_copy(dst_hbm.at[pl.ds(w * DEG_RPT, DEG_RPT), :], dst_v)

    def row(j, _):
        pltpu.sync_copy(ones_v, acc.at[dst_v.at[j]], add=True)
        return 0
    lax.fori_loop(0, DEG_RPT, row, 0)
    plsc.subcore_barrier()

    pltpu.sync_copy(acc.at[pl.ds(s * 640, 640)], out_hbm.at[c, pl.ds(s * 640, 640)])


_sc_degree = functools.partial(
    pl.kernel,
    out_type=jax.ShapeDtypeStruct((NC, DEG_PAD), jnp.float32),
    mesh=plsc.VectorSubcoreMesh(core_axis_name="c", subcore_axis_name="s", num_cores=NC, num_subcores=NS),
    compiler_params=pltpu.CompilerParams(use_tc_tiling_on_sc=False, needs_layout_passes=False),
    scratch_types=[
        pltpu.VMEM((DEG_RPT, DEG_W), jnp.int32),
        pltpu.VMEM((DEG_W,), jnp.float32),
        pltpu.VMEM((640,), jnp.float32),
        pltpu.VMEM_SHARED((DEG_PAD,), jnp.float32),
    ],
)(_sc_degree_body)


NRANGE = 8
RSZ = 1280
ACC_R = 1344
TRASH = 1280
PACK_BITS = 11
PACK = 1 << PACK_BITS
CH = 48
NSUB = EPT // ECH
LCAP = 4160
MAXCH = (ECH + CH - 1) // CH


def _sc_compact_body(src_hbm, dst_hbm, codes_hbm, es, ed, sel):
    c = lax.axis_index("c")
    s = lax.axis_index("s")
    w = c * NS + s

    def edge_chunk(ec, _):
        off = s * EPT + ec * ECH
        pltpu.sync_copy(src_hbm.at[pl.ds(off, ECH)], es)
        pltpu.sync_copy(dst_hbm.at[pl.ds(off, ECH)], ed)

        for rng in range(NRANGE):
            base = rng * RSZ

            def scan_vec(i, cnt):
                sv = es[pl.ds(i * 16, 16)]
                dv = ed[pl.ds(i * 16, 16)]
                m = (dv >= base) & (dv < base + RSZ)
                mi = m.astype(jnp.int32)
                pos = 16 + cnt + plsc.cumsum(mi) - 1
                code = sv * PACK + (dv - base)
                plsc.store_scatter(sel, [pos], code, mask=m)
                return cnt + jnp.sum(mi)
            cnt = lax.fori_loop(0, ECH // 16, scan_vec, jnp.int32(0))

            sel[pl.ds(0, 16)] = jnp.full((16,), 1, jnp.int32) * cnt
            for t in range(CH // 16):
                sel[pl.ds(16 + cnt + t * 16, 16)] = jnp.full(
                    (16,), TRASH, jnp.int32)
            pltpu.sync_copy(sel, codes_hbm.at[rng, w, ec])
        return 0
    lax.fori_loop(0, NSUB, edge_chunk, 0)


_sc_compact = functools.partial(
    pl.kernel,
    out_type=jax.ShapeDtypeStruct((NRANGE, NC * NS, NSUB, LCAP), jnp.int32),
    mesh=plsc.VectorSubcoreMesh(core_axis_name="c", subcore_axis_name="s", num_cores=NC, num_subcores=NS),
    compiler_params=pltpu.CompilerParams(use_tc_tiling_on_sc=False, needs_layout_passes=False),
    scratch_types=[
        pltpu.VMEM((ECH,), jnp.int32),
        pltpu.VMEM((ECH,), jnp.int32),
        pltpu.VMEM((LCAP,), jnp.int32),
    ],
)(_sc_compact_body)


def _sc_scatter_body(F, codes_hbm, tab_hbm, out_hbm,
                     sel, zbuf, rows, idxw, acc, sem_g, sem_s):
    c = lax.axis_index("c")
    s = lax.axis_index("s")

    for zr in range(16):
        def fill_z(col, _, zr=zr):
            zbuf[zr, pl.ds(col * 16, 16)] = jnp.zeros((16,), jnp.float32)
            return 0
        lax.fori_loop(0, F // 16, fill_z, 0)

    def decode(j, r):
        def dec(t, _):
            cv = sel[pl.ds(16 + j * CH + t * 16, 16)]
            idxw[4 + r, pl.ds(t * 16, 16)] = lax.shift_right_logical(
                cv, PACK_BITS)
            idxw[r, pl.ds(t * 16, 16)] = jnp.bitwise_and(cv, PACK - 1)
            return 0
        lax.fori_loop(0, CH // 16, dec, 0)

    def issue_gather(r):
        pltpu.async_copy(tab_hbm.at[idxw.at[4 + r]], rows.at[r], sem_g)

    def wait_gather(r):
        pltpu.make_async_copy(tab_hbm.at[pl.ds(0, CH)], rows.at[r],
                              sem_g).wait()

    def issue_scatter(r):
        pltpu.async_copy(rows.at[r], acc.at[idxw.at[r]], sem_s, add=True)

    def wait_scatter():
        pltpu.make_async_copy(tab_hbm.at[pl.ds(0, CH)], rows.at[0],
                              sem_s).wait()

    for rr in range(NRANGE // NC):
        rng_s0 = rr
        rng_s1 = 4 + rr
        base = (c * (NRANGE // NC) + rr) * RSZ
        limit = jnp.where(c * (NRANGE // NC) + rr == NRANGE - 1,
                          N - (NRANGE - 1) * RSZ, RSZ)

        def zero_k(k, _):
            @pl.when(k % NS == s)
            def _():
                pltpu.sync_copy(zbuf, acc.at[pl.ds(k * 16, 16), :])
            return 0
        lax.fori_loop(0, ACC_R // 16, zero_k, 0)
        plsc.subcore_barrier()

        def sublist(t2, _):
            plist = 2 * s + (t2 // NSUB)
            sub = t2 % NSUB

            @pl.when(c == 0)
            def _():
                pltpu.sync_copy(codes_hbm.at[rng_s0, plist, sub], sel)

            @pl.when(c == 1)
            def _():
                pltpu.sync_copy(codes_hbm.at[rng_s1, plist, sub], sel)

            cnt = lax.shift_right_logical(jnp.sum(sel[pl.ds(0, 16)]), 4)
            nch = (cnt + (CH - 1)) // CH

            @pl.when(nch > 0)
            def _():
                decode(jnp.int32(0), 0)
                issue_gather(0)

            @pl.when(nch > 1)
            def _():
                decode(jnp.int32(1), 1)
                issue_gather(1)

            def chunk3_loop(jj, _):
                for r in (0, 1, 2):
                    j = 3 * jj + r

                    @pl.when(j < nch)
                    def _(j=j, r=r):
                        wait_gather(r)
                        issue_scatter(r)

                        @pl.when(j + 2 < nch)
                        def _(j=j, r=r):
                            @pl.when(j >= 1)
                            def _():
                                wait_scatter()
                            r2 = (r + 2) % 3
                            decode(j + 2, r2)
                            issue_gather(r2)
                return 0
            lax.fori_loop(0, (MAXCH + 2) // 3, chunk3_loop, 0)

            for t in range(3):
                @pl.when(nch >= t + 1)
                def _():
                    wait_scatter()
            return 0
        lax.fori_loop(0, 2 * NSUB, sublist, 0)
        plsc.subcore_barrier()

        def wb(k, _):
            @pl.when((k % NS == s) & (k * 16 < limit))
            def _():
                pltpu.sync_copy(acc.at[pl.ds(k * 16, 16), :],
                                out_hbm.at[pl.ds(base + k * 16, 16), :])
            return 0
        lax.fori_loop(0, RSZ // 16, wb, 0)
        if rr != NRANGE // NC - 1:
            plsc.subcore_barrier()


def _make_scatter(F):
    return functools.partial(
        pl.kernel,
        out_type=jax.ShapeDtypeStruct((N, F), jnp.float32),
        mesh=plsc.VectorSubcoreMesh(core_axis_name="c", subcore_axis_name="s", num_cores=NC, num_subcores=NS),
        compiler_params=pltpu.CompilerParams(use_tc_tiling_on_sc=False, needs_layout_passes=False),
        scratch_types=[
            pltpu.VMEM((LCAP,), jnp.int32),
            pltpu.VMEM((16, F), jnp.float32),
            pltpu.VMEM((3, CH, F), jnp.float32),
            pltpu.VMEM((8, CH), jnp.int32),
            pltpu.VMEM_SHARED((ACC_R, F), jnp.float32),
            pltpu.SemaphoreType.DMA,
            pltpu.SemaphoreType.DMA,
        ],
    )(functools.partial(_sc_scatter_body, F))


_sc_scatter_h = _make_scatter(H)
_sc_scatter_e = _make_scatter(ENC)


BPAD = 100352
QT = BPAD // 32
C2 = 56
NCH2 = QT // C2


def _sc_sqdist_body(ein_hbm, eout_hbm, emb_hbm, out_hbm,
                    ei, eo, ri, ro, sem):
    c = lax.axis_index("c")
    s = lax.axis_index("s")
    w = c * NS + s
    qbase = w * QT

    pltpu.sync_copy(ein_hbm.at[pl.ds(qbase, QT)], ei)
    pltpu.sync_copy(eout_hbm.at[pl.ds(qbase, QT)], eo)

    pltpu.async_copy(emb_hbm.at[ei.at[pl.ds(0, C2)]], ri.at[0], sem)
    pltpu.async_copy(emb_hbm.at[eo.at[pl.ds(0, C2)]], ro.at[0], sem)

    def chunk(j, _):
        r = jnp.bitwise_and(j, 1)
        pltpu.make_async_copy(emb_hbm.at[pl.ds(0, C2)], ri.at[r], sem).wait()
        pltpu.make_async_copy(emb_hbm.at[pl.ds(0, C2)], ro.at[r], sem).wait()

        @pl.when(j + 1 < NCH2)
        def _():
            pltpu.async_copy(emb_hbm.at[ei.at[pl.ds((j + 1) * C2, C2)]],
                             ri.at[1 - r], sem)
            pltpu.async_copy(emb_hbm.at[eo.at[pl.ds((j + 1) * C2, C2)]],
                             ro.at[1 - r], sem)

        def rowfn(a, _):
            for bcol in range(ENC // 16):
                x = ri[r, a, pl.ds(bcol * 16, 16)]
                y = ro[r, a, pl.ds(bcol * 16, 16)]
                d = x - y
                ri[r, a, pl.ds(bcol * 16, 16)] = d * d
            return 0
        lax.fori_loop(0, C2, rowfn, 0)
        pltpu.sync_copy(ri.at[r], out_hbm.at[pl.ds(qbase + j * C2, C2), :])
        return 0
    lax.fori_loop(0, NCH2, chunk, 0)


_sc_sqdist = functools.partial(
    pl.kernel,
    out_type=jax.ShapeDtypeStruct((BPAD, ENC), jnp.float32),
    mesh=plsc.VectorSubcoreMesh(core_axis_name="c", subcore_axis_name="s", num_cores=NC, num_subcores=NS),
        compiler_params=pltpu.CompilerParams(use_tc_tiling_on_sc=False, needs_layout_passes=False),
    scratch_types=[
        pltpu.VMEM((QT,), jnp.int32),
        pltpu.VMEM((QT,), jnp.int32),
        pltpu.VMEM((2, C2, ENC), jnp.float32),
        pltpu.VMEM((2, C2, ENC), jnp.float32),
        pltpu.SemaphoreType.DMA,
    ],
)(_sc_sqdist_body)


BM = 1000


def _tc_enc1_body(x_ref, w1_ref, d0_ref, d1_ref, xt1_ref, dinv_ref):
    dv = lax.rsqrt(d0_ref[...] + d1_ref[...] + 1.0)
    xw = jnp.dot(x_ref[...], w1_ref[...], preferred_element_type=jnp.float32)
    xt1_ref[...] = xw * dv
    dinv_ref[...] = dv


def _tc_enc1(x, w1, d0, d1):
    return pl.pallas_call(
        _tc_enc1_body,
        grid=(N // BM,),
        in_specs=[
            pl.BlockSpec((BM, D), lambda i: (i, 0)),
            pl.BlockSpec((D, H), lambda i: (0, 0)),
            pl.BlockSpec((BM, 1), lambda i: (i, 0)),
            pl.BlockSpec((BM, 1), lambda i: (i, 0)),
        ],
        out_specs=[
            pl.BlockSpec((BM, H), lambda i: (i, 0)),
            pl.BlockSpec((BM, 1), lambda i: (i, 0)),
        ],
        out_shape=[
            jax.ShapeDtypeStruct((N, H), jnp.float32),
            jax.ShapeDtypeStruct((N, 1), jnp.float32),
        ],
    )(x, w1, d0, d1)


def _tc_enc2_body(s1_ref, xt1_ref, dinv_ref, b1_ref, w2_ref, xt2_ref):
    h = (s1_ref[...] + xt1_ref[...]) * dinv_ref[...] + b1_ref[...]
    h = jnp.maximum(h, 0.0)
    xw = jnp.dot(h, w2_ref[...], preferred_element_type=jnp.float32)
    xt2_ref[...] = xw * dinv_ref[...]


def _tc_enc2(s1, xt1, dinv, b1, w2):
    return pl.pallas_call(
        _tc_enc2_body,
        grid=(N // BM,),
        in_specs=[
            pl.BlockSpec((BM, H), lambda i: (i, 0)),
            pl.BlockSpec((BM, H), lambda i: (i, 0)),
            pl.BlockSpec((BM, 1), lambda i: (i, 0)),
            pl.BlockSpec((1, H), lambda i: (0, 0)),
            pl.BlockSpec((H, ENC), lambda i: (0, 0)),
        ],
        out_specs=pl.BlockSpec((BM, ENC), lambda i: (i, 0)),
        out_shape=jax.ShapeDtypeStruct((N, ENC), jnp.float32),
    )(s1, xt1, dinv, b1, w2)


def _tc_emb_body(s2_ref, xt2_ref, dinv_ref, b2_ref, emb_ref):
    z = (s2_ref[...] + xt2_ref[...]) * dinv_ref[...] + b2_ref[...]
    emb_ref[...] = jnp.maximum(z, 0.0)


def _tc_emb(s2, xt2, dinv, b2):
    return pl.pallas_call(
        _tc_emb_body,
        grid=(N // BM,),
        in_specs=[
            pl.BlockSpec((BM, ENC), lambda i: (i, 0)),
            pl.BlockSpec((BM, ENC), lambda i: (i, 0)),
            pl.BlockSpec((BM, 1), lambda i: (i, 0)),
            pl.BlockSpec((1, ENC), lambda i: (0, 0)),
        ],
        out_specs=pl.BlockSpec((BM, ENC), lambda i: (i, 0)),
        out_shape=jax.ShapeDtypeStruct((N, ENC), jnp.float32),
    )(s2, xt2, dinv, b2)


BD = 2000


def _tc_dec_body(sq_ref, plh_ref, wa_ref, wb_ref, db1_ref, dw2_ref, db2_ref,
                 out_ref):
    z = (jnp.dot(sq_ref[...], wa_ref[...], preferred_element_type=jnp.float32)
         + jnp.dot(plh_ref[...], wb_ref[...],
                   preferred_element_type=jnp.float32)
         + db1_ref[...])
    z = jnp.where(z >= 0.0, z, 0.1 * z)
    z = jnp.dot(z, dw2_ref[...], preferred_element_type=jnp.float32) \
        + db2_ref[...]
    z = jnp.clip(jnp.abs(z), 0.0, 40.0)
    out_ref[...] = 1.0 / (jnp.exp((z - 2.0) * 2.0) + 1.0)


def _tc_dec(sq, plh, wa, wb, db1, dw2, db2):
    return pl.pallas_call(
        _tc_dec_body,
        grid=(B // BD,),
        in_specs=[
            pl.BlockSpec((BD, ENC), lambda i: (i, 0)),
            pl.BlockSpec((BD, PLH), lambda i: (i, 0)),
            pl.BlockSpec((ENC, MLP), lambda i: (0, 0)),
            pl.BlockSpec((PLH, MLP), lambda i: (0, 0)),
            pl.BlockSpec((1, MLP), lambda i: (0, 0)),
            pl.BlockSpec((MLP, 1), lambda i: (0, 0)),
            pl.BlockSpec((1, 1), lambda i: (0, 0)),
        ],
        out_specs=pl.BlockSpec((BD, 1), lambda i: (i, 0)),
        out_shape=jax.ShapeDtypeStruct((B, 1), jnp.float32),
    )(sq, plh, wa, wb, db1, dw2, db2)


def kernel(node_x, gnn_edge_index, edges, plh_x, W1, b1, W2, b2,
           dW1, db1, dW2, db2):
    src = gnn_edge_index[0]
    dst = gnn_edge_index[1]
    srcp = jnp.pad(src, (0, EPAD - E))
    dstp_e = jnp.pad(dst, (0, EPAD - E), constant_values=DST_PAD)

    dstp = jnp.concatenate(
        [dst, jnp.full((NC * NS * DEG_RPT * DEG_W - E,), N, jnp.int32)]
    ).reshape(NC * NS * DEG_RPT, DEG_W)
    degp = _sc_degree(dstp)
    d0 = degp[0, :N].reshape(N, 1)
    d1 = degp[1, :N].reshape(N, 1)

    xt1, dinv = _tc_enc1(node_x, W1, d0, d1)
    codes = _sc_compact(srcp, dstp_e)
    s1 = _sc_scatter_h(codes, xt1)
    xt2 = _tc_enc2(s1, xt1, dinv, b1.reshape(1, H), W2)
    s2 = _sc_scatter_e(codes, xt2)
    emb = _tc_emb(s2, xt2, dinv, b2.reshape(1, ENC))

    ein = jnp.pad(edges[:, 0], (0, BPAD - B))
    eout = jnp.pad(edges[:, 1], (0, BPAD - B))
    sq = _sc_sqdist(ein, eout, emb)

    out = _tc_dec(sq, plh_x, dW1[:ENC], dW1[ENC:], db1.reshape(1, MLP),
                  dW2, db2.reshape(1, 1))
    return out.reshape(-1)


# speedup vs baseline: 6.1349x; 1.6543x over previous
import functools

import jax
import jax.numpy as jnp
from jax import lax
from jax.experimental import pallas as pl
from jax.experimental.pallas import tpu as pltpu
from jax.experimental.pallas import tpu_sc as plsc

N = 10000
E = 320000
D = 128
H = 512
ENC = 256
PLH = 64
B = 100000
MLP = 25

NC, NS = 2, 16
ECH = 4096
EPT = 3 * ECH
EPAD = NC * NS * EPT
DST_PAD = 16384


DEG_W = 128
DEG_RPT = 80
DEG_PAD = 10240


def _sc_degree_body(dst_hbm, out_hbm, dst_v, ones_v, zbuf, acc):
    c = lax.axis_index("c")
    s = lax.axis_index("s")
    w = c * NS + s

    def fill_z(k, _):
        zbuf[pl.ds(k * 16, 16)] = jnp.zeros((16,), jnp.float32)
        return 0
    lax.fori_loop(0, 40, fill_z, 0)
    for off in range(0, DEG_W, 16):
        ones_v[pl.ds(off, 16)] = jnp.ones((16,), jnp.float32)

    pltpu.sync_copy(zbuf, acc.at[pl.ds(s * 640, 640)])
    plsc.subcore_barrier()

    pltpu.sync_copy(dst_hbm.at[pl.ds(w * DEG_RPT, DEG_RPT), :], dst_v)

    def row(j, _):
        pltpu.sync_copy(ones_v, acc.at[dst_v.at[j]], add=True)
        return 0
    lax.fori_loop(0, DEG_RPT, row, 0)
    plsc.subcore_barrier()

    pltpu.sync_copy(acc.at[pl.ds(s * 640, 640)], out_hbm.at[c, pl.ds(s * 640, 640)])


_sc_degree = functools.partial(
    pl.kernel,
    out_type=jax.ShapeDtypeStruct((NC, DEG_PAD), jnp.float32),
    mesh=plsc.VectorSubcoreMesh(core_axis_name="c", subcore_axis_name="s", num_cores=NC, num_subcores=NS),
    compiler_params=pltpu.CompilerParams(use_tc_tiling_on_sc=False, needs_layout_passes=False),
    scratch_types=[
        pltpu.VMEM((DEG_RPT, DEG_W), jnp.int32),
        pltpu.VMEM((DEG_W,), jnp.float32),
        pltpu.VMEM((640,), jnp.float32),
        pltpu.VMEM_SHARED((DEG_PAD,), jnp.float32),
    ],
)(_sc_degree_body)


NRANGE = 8
RSZ = 1280
ACC_R = 1344
TRASH = 1280
PACK_BITS = 11
PACK = 1 << PACK_BITS
CH = 48
NSUB = EPT // ECH
LCAP = 4160
MAXCH = (ECH + CH - 1) // CH


def _sc_compact_body(src_hbm, dst_hbm, codes_hbm, es, ed, sel):
    c = lax.axis_index("c")
    s = lax.axis_index("s")
    w = c * NS + s

    def edge_chunk(ec, _):
        off = w * EPT + ec * ECH
        pltpu.sync_copy(src_hbm.at[pl.ds(off, ECH)], es)
        pltpu.sync_copy(dst_hbm.at[pl.ds(off, ECH)], ed)

        for rng in range(NRANGE):
            base = rng * RSZ

            def scan_vec(i, cnt):
                sv = es[pl.ds(i * 16, 16)]
                dv = ed[pl.ds(i * 16, 16)]
                m = (dv >= base) & (dv < base + RSZ)
                mi = m.astype(jnp.int32)
                pos = 16 + cnt + plsc.cumsum(mi) - 1
                code = sv * PACK + (dv - base)
                plsc.store_scatter(sel, [pos], code, mask=m)
                return cnt + jnp.sum(mi)
            cnt = lax.fori_loop(0, ECH // 16, scan_vec, jnp.int32(0))

            sel[pl.ds(0, 16)] = jnp.full((16,), 1, jnp.int32) * cnt
            for t in range(CH // 16):
                sel[pl.ds(16 + cnt + t * 16, 16)] = jnp.full(
                    (16,), TRASH, jnp.int32)
            pltpu.sync_copy(sel, codes_hbm.at[rng, w, ec])
        return 0
    lax.fori_loop(0, NSUB, edge_chunk, 0)


_sc_compact = functools.partial(
    pl.kernel,
    out_type=jax.ShapeDtypeStruct((NRANGE, NC * NS, NSUB, LCAP), jnp.int32),
    mesh=plsc.VectorSubcoreMesh(core_axis_name="c", subcore_axis_name="s", num_cores=NC, num_subcores=NS),
    compiler_params=pltpu.CompilerParams(use_tc_tiling_on_sc=False, needs_layout_passes=False),
    scratch_types=[
        pltpu.VMEM((ECH,), jnp.int32),
        pltpu.VMEM((ECH,), jnp.int32),
        pltpu.VMEM((LCAP,), jnp.int32),
    ],
)(_sc_compact_body)


def _sc_scatter_body(F, codes_hbm, tab_hbm, out_hbm,
                     sel, zbuf, rows, idxw, acc, sem_g, sem_s):
    c = lax.axis_index("c")
    s = lax.axis_index("s")

    for zr in range(16):
        def fill_z(col, _, zr=zr):
            zbuf[zr, pl.ds(col * 16, 16)] = jnp.zeros((16,), jnp.float32)
            return 0
        lax.fori_loop(0, F // 16, fill_z, 0)

    def decode(j, r):
        def dec(t, _):
            cv = sel[pl.ds(16 + j * CH + t * 16, 16)]
            idxw[4 + r, pl.ds(t * 16, 16)] = lax.shift_right_logical(
                cv, PACK_BITS)
            idxw[r, pl.ds(t * 16, 16)] = jnp.bitwise_and(cv, PACK - 1)
            return 0
        lax.fori_loop(0, CH // 16, dec, 0)

    def issue_gather(r):
        pltpu.async_copy(tab_hbm.at[idxw.at[4 + r]], rows.at[r], sem_g)

    def wait_gather(r):
        pltpu.make_async_copy(tab_hbm.at[pl.ds(0, CH)], rows.at[r],
                              sem_g).wait()

    def issue_scatter(r):
        pltpu.async_copy(rows.at[r], acc.at[idxw.at[r]], sem_s, add=True)

    def wait_scatter():
        pltpu.make_async_copy(tab_hbm.at[pl.ds(0, CH)], rows.at[0],
                              sem_s).wait()

    for rr in range(NRANGE // NC):
        rng_s0 = rr
        rng_s1 = 4 + rr
        base = (c * (NRANGE // NC) + rr) * RSZ
        limit = jnp.where(c * (NRANGE // NC) + rr == NRANGE - 1,
                          N - (NRANGE - 1) * RSZ, RSZ)

        def zero_k(k, _):
            @pl.when(k % NS == s)
            def _():
                pltpu.sync_copy(zbuf, acc.at[pl.ds(k * 16, 16), :])
            return 0
        lax.fori_loop(0, ACC_R // 16, zero_k, 0)
        plsc.subcore_barrier()

        def sublist(t2, _):
            plist = 2 * s + (t2 // NSUB)
            sub = t2 % NSUB

            @pl.when(c == 0)
            def _():
                pltpu.sync_copy(codes_hbm.at[rng_s0, plist, sub], sel)

            @pl.when(c == 1)
            def _():
                pltpu.sync_copy(codes_hbm.at[rng_s1, plist, sub], sel)

            cnt = lax.shift_right_logical(jnp.sum(sel[pl.ds(0, 16)]), 4)
            nch = (cnt + (CH - 1)) // CH

            @pl.when(nch > 0)
            def _():
                decode(jnp.int32(0), 0)
                issue_gather(0)

            @pl.when(nch > 1)
            def _():
                decode(jnp.int32(1), 1)
                issue_gather(1)

            def chunk3_loop(jj, _):
                for r in (0, 1, 2):
                    j = 3 * jj + r

                    @pl.when(j < nch)
                    def _(j=j, r=r):
                        wait_gather(r)
                        issue_scatter(r)

                        @pl.when(j + 2 < nch)
                        def _(j=j, r=r):
                            @pl.when(j >= 1)
                            def _():
                                wait_scatter()
                            r2 = (r + 2) % 3
                            decode(j + 2, r2)
                            issue_gather(r2)
                return 0
            lax.fori_loop(0, (MAXCH + 2) // 3, chunk3_loop, 0)

            for t in range(3):
                @pl.when(nch >= t + 1)
                def _():
                    wait_scatter()
            return 0
        lax.fori_loop(0, 2 * NSUB, sublist, 0)
        plsc.subcore_barrier()

        def wb(k, _):
            @pl.when((k % NS == s) & (k * 16 < limit))
            def _():
                pltpu.sync_copy(acc.at[pl.ds(k * 16, 16), :],
                                out_hbm.at[pl.ds(base + k * 16, 16), :])
            return 0
        lax.fori_loop(0, RSZ // 16, wb, 0)
        if rr != NRANGE // NC - 1:
            plsc.subcore_barrier()


def _make_scatter(F):
    return functools.partial(
        pl.kernel,
        out_type=jax.ShapeDtypeStruct((N, F), jnp.float32),
        mesh=plsc.VectorSubcoreMesh(core_axis_name="c", subcore_axis_name="s", num_cores=NC, num_subcores=NS),
        compiler_params=pltpu.CompilerParams(use_tc_tiling_on_sc=False, needs_layout_passes=False),
        scratch_types=[
            pltpu.VMEM((LCAP,), jnp.int32),
            pltpu.VMEM((16, F), jnp.float32),
            pltpu.VMEM((3, CH, F), jnp.float32),
            pltpu.VMEM((8, CH), jnp.int32),
            pltpu.VMEM_SHARED((ACC_R, F), jnp.float32),
            pltpu.SemaphoreType.DMA,
            pltpu.SemaphoreType.DMA,
        ],
    )(functools.partial(_sc_scatter_body, F))


_sc_scatter_h = _make_scatter(H)
_sc_scatter_e = _make_scatter(ENC)


BPAD = 100352
QT = BPAD // 32
C2 = 56
NCH2 = QT // C2


def _sc_sqdist_body(ein_hbm, eout_hbm, emb_hbm, out_hbm,
                    ei, eo, ri, ro, sem):
    c = lax.axis_index("c")
    s = lax.axis_index("s")
    w = c * NS + s
    qbase = w * QT

    pltpu.sync_copy(ein_hbm.at[pl.ds(qbase, QT)], ei)
    pltpu.sync_copy(eout_hbm.at[pl.ds(qbase, QT)], eo)

    pltpu.async_copy(emb_hbm.at[ei.at[pl.ds(0, C2)]], ri.at[0], sem)
    pltpu.async_copy(emb_hbm.at[eo.at[pl.ds(0, C2)]], ro.at[0], sem)

    def chunk(j, _):
        r = jnp.bitwise_and(j, 1)
        pltpu.make_async_copy(emb_hbm.at[pl.ds(0, C2)], ri.at[r], sem).wait()
        pltpu.make_async_copy(emb_hbm.at[pl.ds(0, C2)], ro.at[r], sem).wait()

        @pl.when(j + 1 < NCH2)
        def _():
            pltpu.async_copy(emb_hbm.at[ei.at[pl.ds((j + 1) * C2, C2)]],
                             ri.at[1 - r], sem)
            pltpu.async_copy(emb_hbm.at[eo.at[pl.ds((j + 1) * C2, C2)]],
                             ro.at[1 - r], sem)

        def rowfn(a, _):
            for bcol in range(ENC // 16):
                x = ri[r, a, pl.ds(bcol * 16, 16)]
                y = ro[r, a, pl.ds(bcol * 16, 16)]
                d = x - y
                ri[r, a, pl.ds(bcol * 16, 16)] = d * d
            return 0
        lax.fori_loop(0, C2, rowfn, 0)
        pltpu.sync_copy(ri.at[r], out_hbm.at[pl.ds(qbase + j * C2, C2), :])
        return 0
    lax.fori_loop(0, NCH2, chunk, 0)


_sc_sqdist = functools.partial(
    pl.kernel,
    out_type=jax.ShapeDtypeStruct((BPAD, ENC), jnp.float32),
    mesh=plsc.VectorSubcoreMesh(core_axis_name="c", subcore_axis_name="s", num_cores=NC, num_subcores=NS),
        compiler_params=pltpu.CompilerParams(use_tc_tiling_on_sc=False, needs_layout_passes=False),
    scratch_types=[
        pltpu.VMEM((QT,), jnp.int32),
        pltpu.VMEM((QT,), jnp.int32),
        pltpu.VMEM((2, C2, ENC), jnp.float32),
        pltpu.VMEM((2, C2, ENC), jnp.float32),
        pltpu.SemaphoreType.DMA,
    ],
)(_sc_sqdist_body)


BM = 1000


def _tc_enc1_body(x_ref, w1_ref, d0_ref, d1_ref, xt1_ref, dinv_ref):
    dv = lax.rsqrt(d0_ref[...] + d1_ref[...] + 1.0)
    xw = jnp.dot(x_ref[...], w1_ref[...], preferred_element_type=jnp.float32)
    xt1_ref[...] = xw * dv
    dinv_ref[...] = dv


def _tc_enc1(x, w1, d0, d1):
    return pl.pallas_call(
        _tc_enc1_body,
        grid=(N // BM,),
        in_specs=[
            pl.BlockSpec((BM, D), lambda i: (i, 0)),
            pl.BlockSpec((D, H), lambda i: (0, 0)),
            pl.BlockSpec((BM, 1), lambda i: (i, 0)),
            pl.BlockSpec((BM, 1), lambda i: (i, 0)),
        ],
        out_specs=[
            pl.BlockSpec((BM, H), lambda i: (i, 0)),
            pl.BlockSpec((BM, 1), lambda i: (i, 0)),
        ],
        out_shape=[
            jax.ShapeDtypeStruct((N, H), jnp.float32),
            jax.ShapeDtypeStruct((N, 1), jnp.float32),
        ],
    )(x, w1, d0, d1)


def _tc_enc2_body(s1_ref, xt1_ref, dinv_ref, b1_ref, w2_ref, xt2_ref):
    h = (s1_ref[...] + xt1_ref[...]) * dinv_ref[...] + b1_ref[...]
    h = jnp.maximum(h, 0.0)
    xw = jnp.dot(h, w2_ref[...], preferred_element_type=jnp.float32)
    xt2_ref[...] = xw * dinv_ref[...]


def _tc_enc2(s1, xt1, dinv, b1, w2):
    return pl.pallas_call(
        _tc_enc2_body,
        grid=(N // BM,),
        in_specs=[
            pl.BlockSpec((BM, H), lambda i: (i, 0)),
            pl.BlockSpec((BM, H), lambda i: (i, 0)),
            pl.BlockSpec((BM, 1), lambda i: (i, 0)),
            pl.BlockSpec((1, H), lambda i: (0, 0)),
            pl.BlockSpec((H, ENC), lambda i: (0, 0)),
        ],
        out_specs=pl.BlockSpec((BM, ENC), lambda i: (i, 0)),
        out_shape=jax.ShapeDtypeStruct((N, ENC), jnp.float32),
    )(s1, xt1, dinv, b1, w2)


def _tc_emb_body(s2_ref, xt2_ref, dinv_ref, b2_ref, emb_ref):
    z = (s2_ref[...] + xt2_ref[...]) * dinv_ref[...] + b2_ref[...]
    emb_ref[...] = jnp.maximum(z, 0.0)


def _tc_emb(s2, xt2, dinv, b2):
    return pl.pallas_call(
        _tc_emb_body,
        grid=(N // BM,),
        in_specs=[
            pl.BlockSpec((BM, ENC), lambda i: (i, 0)),
            pl.BlockSpec((BM, ENC), lambda i: (i, 0)),
            pl.BlockSpec((BM, 1), lambda i: (i, 0)),
            pl.BlockSpec((1, ENC), lambda i: (0, 0)),
        ],
        out_specs=pl.BlockSpec((BM, ENC), lambda i: (i, 0)),
        out_shape=jax.ShapeDtypeStruct((N, ENC), jnp.float32),
    )(s2, xt2, dinv, b2)


BD = 2000


def _tc_dec_body(sq_ref, plh_ref, wa_ref, wb_ref, db1_ref, dw2_ref, db2_ref,
                 out_ref):
    z = (jnp.dot(sq_ref[...], wa_ref[...], preferred_element_type=jnp.float32)
         + jnp.dot(plh_ref[...], wb_ref[...],
                   preferred_element_type=jnp.float32)
         + db1_ref[...])
    z = jnp.where(z >= 0.0, z, 0.1 * z)
    z = jnp.dot(z, dw2_ref[...], preferred_element_type=jnp.float32) \
        + db2_ref[...]
    z = jnp.clip(jnp.abs(z), 0.0, 40.0)
    out_ref[...] = 1.0 / (jnp.exp((z - 2.0) * 2.0) + 1.0)


def _tc_dec(sq, plh, wa, wb, db1, dw2, db2):
    return pl.pallas_call(
        _tc_dec_body,
        grid=(B // BD,),
        in_specs=[
            pl.BlockSpec((BD, ENC), lambda i: (i, 0)),
            pl.BlockSpec((BD, PLH), lambda i: (i, 0)),
            pl.BlockSpec((ENC, MLP), lambda i: (0, 0)),
            pl.BlockSpec((PLH, MLP), lambda i: (0, 0)),
            pl.BlockSpec((1, MLP), lambda i: (0, 0)),
            pl.BlockSpec((MLP, 1), lambda i: (0, 0)),
            pl.BlockSpec((1, 1), lambda i: (0, 0)),
        ],
        out_specs=pl.BlockSpec((BD, 1), lambda i: (i, 0)),
        out_shape=jax.ShapeDtypeStruct((B, 1), jnp.float32),
    )(sq, plh, wa, wb, db1, dw2, db2)


def kernel(node_x, gnn_edge_index, edges, plh_x, W1, b1, W2, b2,
           dW1, db1, dW2, db2):
    src = gnn_edge_index[0]
    dst = gnn_edge_index[1]
    srcp = jnp.pad(src, (0, EPAD - E))
    dstp_e = jnp.pad(dst, (0, EPAD - E), constant_values=DST_PAD)

    dstp = jnp.concatenate(
        [dst, jnp.full((NC * NS * DEG_RPT * DEG_W - E,), N, jnp.int32)]
    ).reshape(NC * NS * DEG_RPT, DEG_W)
    degp = _sc_degree(dstp)
    d0 = degp[0, :N].reshape(N, 1)
    d1 = degp[1, :N].reshape(N, 1)

    xt1, dinv = _tc_enc1(node_x, W1, d0, d1)
    codes = _sc_compact(srcp, dstp_e)
    s1 = _sc_scatter_h(codes, xt1)
    xt2 = _tc_enc2(s1, xt1, dinv, b1.reshape(1, H), W2)
    s2 = _sc_scatter_e(codes, xt2)
    emb = _tc_emb(s2, xt2, dinv, b2.reshape(1, ENC))

    ein = jnp.pad(edges[:, 0], (0, BPAD - B))
    eout = jnp.pad(edges[:, 1], (0, BPAD - B))
    sq = _sc_sqdist(ein, eout, emb)

    out = _tc_dec(sq, plh_x, dW1[:ENC], dW1[ENC:], db1.reshape(1, MLP),
                  dW2, db2.reshape(1, 1))
    return out.reshape(-1)


# speedup vs baseline: 6.1373x; 1.0004x over previous
import functools

import jax
import jax.numpy as jnp
from jax import lax
from jax.experimental import pallas as pl
from jax.experimental.pallas import tpu as pltpu
from jax.experimental.pallas import tpu_sc as plsc

N = 10000
E = 320000
D = 128
H = 512
ENC = 256
PLH = 64
B = 100000
MLP = 25

NC, NS = 2, 16
ECH = 4096
EPT = 3 * ECH
EPAD = NC * NS * EPT
DST_PAD = 16384


DEG_W = 128
DEG_RPT = 80
DEG_PAD = 10240


def _sc_degree_body(dst_hbm, out_hbm, dst_v, ones_v, zbuf, acc):
    c = lax.axis_index("c")
    s = lax.axis_index("s")
    w = c * NS + s

    def fill_z(k, _):
        zbuf[pl.ds(k * 16, 16)] = jnp.zeros((16,), jnp.float32)
        return 0
    lax.fori_loop(0, 40, fill_z, 0)
    for off in range(0, DEG_W, 16):
        ones_v[pl.ds(off, 16)] = jnp.ones((16,), jnp.float32)

    pltpu.sync_copy(zbuf, acc.at[pl.ds(s * 640, 640)])
    plsc.subcore_barrier()

    pltpu.sync_copy(dst_hbm.at[pl.ds(w * DEG_RPT, DEG_RPT), :], dst_v)

    def row(j, _):
        pltpu.sync_copy(ones_v, acc.at[dst_v.at[j]], add=True)
        return 0
    lax.fori_loop(0, DEG_RPT, row, 0)
    plsc.subcore_barrier()

    pltpu.sync_copy(acc.at[pl.ds(s * 640, 640)], out_hbm.at[c, pl.ds(s * 640, 640)])


_sc_degree = functools.partial(
    pl.kernel,
    out_type=jax.ShapeDtypeStruct((NC, DEG_PAD), jnp.float32),
    mesh=plsc.VectorSubcoreMesh(core_axis_name="c", subcore_axis_name="s", num_cores=NC, num_subcores=NS),
    compiler_params=pltpu.CompilerParams(use_tc_tiling_on_sc=False, needs_layout_passes=False),
    scratch_types=[
        pltpu.VMEM((DEG_RPT, DEG_W), jnp.int32),
        pltpu.VMEM((DEG_W,), jnp.float32),
        pltpu.VMEM((640,), jnp.float32),
        pltpu.VMEM_SHARED((DEG_PAD,), jnp.float32),
    ],
)(_sc_degree_body)


NRANGE = 8
RSZ = 1280
ACC_R = 1344
TRASH = 1280
PACK_BITS = 11
PACK = 1 << PACK_BITS
CH = 48
NSUB = EPT // ECH
LCAP = 4160
MAXCH = (ECH + CH - 1) // CH


def _sc_compact_body(src_hbm, dst_hbm, codes_hbm, es, ed, sel):
    c = lax.axis_index("c")
    s = lax.axis_index("s")
    w = c * NS + s

    def edge_chunk(ec, _):
        off = w * EPT + ec * ECH
        pltpu.sync_copy(src_hbm.at[pl.ds(off, ECH)], es)
        pltpu.sync_copy(dst_hbm.at[pl.ds(off, ECH)], ed)

        for rng in range(NRANGE):
            base = rng * RSZ

            def scan_vec(i, cnt):
                sv = es[pl.ds(i * 16, 16)]
                dv = ed[pl.ds(i * 16, 16)]
                m = (dv >= base) & (dv < base + RSZ)
                mi = m.astype(jnp.int32)
                pos = 16 + cnt + plsc.cumsum(mi) - 1
                code = sv * PACK + (dv - base)
                plsc.store_scatter(sel, [pos], code, mask=m)
                return cnt + jnp.sum(mi)
            cnt = lax.fori_loop(0, ECH // 16, scan_vec, jnp.int32(0))

            sel[pl.ds(0, 16)] = jnp.full((16,), 1, jnp.int32) * cnt
            for t in range(CH // 16):
                sel[pl.ds(16 + cnt + t * 16, 16)] = jnp.full(
                    (16,), TRASH, jnp.int32)
            pltpu.sync_copy(sel, codes_hbm.at[rng, w, ec])
        return 0
    lax.fori_loop(0, NSUB, edge_chunk, 0)


_sc_compact = functools.partial(
    pl.kernel,
    out_type=jax.ShapeDtypeStruct((NRANGE, NC * NS, NSUB, LCAP), jnp.int32),
    mesh=plsc.VectorSubcoreMesh(core_axis_name="c", subcore_axis_name="s", num_cores=NC, num_subcores=NS),
    compiler_params=pltpu.CompilerParams(use_tc_tiling_on_sc=False, needs_layout_passes=False),
    scratch_types=[
        pltpu.VMEM((ECH,), jnp.int32),
        pltpu.VMEM((ECH,), jnp.int32),
        pltpu.VMEM((LCAP,), jnp.int32),
    ],
)(_sc_compact_body)


def _sc_scatter_body(F, codes_hbm, tab_hbm, out_hbm,
                     sel, zbuf, rows, idxw, acc, sem_g, sem_s):
    c = lax.axis_index("c")
    s = lax.axis_index("s")

    for zr in range(16):
        def fill_z(col, _, zr=zr):
            zbuf[zr, pl.ds(col * 16, 16)] = jnp.zeros((16,), jnp.float32)
            return 0
        lax.fori_loop(0, F // 16, fill_z, 0)

    def decode(j, r):
        def dec(t, _):
            cv = sel[pl.ds(16 + j * CH + t * 16, 16)]
            idxw[4 + r, pl.ds(t * 16, 16)] = lax.shift_right_logical(
                cv, PACK_BITS)
            idxw[r, pl.ds(t * 16, 16)] = jnp.bitwise_and(cv, PACK - 1)
            return 0
        lax.fori_loop(0, CH // 16, dec, 0)

    def issue_gather(r):
        pltpu.async_copy(tab_hbm.at[idxw.at[4 + r]], rows.at[r], sem_g)

    def wait_gather(r):
        pltpu.make_async_copy(tab_hbm.at[pl.ds(0, CH)], rows.at[r],
                              sem_g).wait()

    def issue_scatter(r):
        pltpu.async_copy(rows.at[r], acc.at[idxw.at[r]], sem_s, add=True)

    def wait_scatter():
        pltpu.make_async_copy(tab_hbm.at[pl.ds(0, CH)], rows.at[0],
                              sem_s).wait()

    for rr in range(NRANGE // NC):
        rng_s0 = rr
        rng_s1 = 4 + rr
        base = (c * (NRANGE // NC) + rr) * RSZ
        limit = jnp.where(c * (NRANGE // NC) + rr == NRANGE - 1,
                          N - (NRANGE - 1) * RSZ, RSZ)

        def zero_k(k, _):
            @pl.when(k % NS == s)
            def _():
                pltpu.sync_copy(zbuf, acc.at[pl.ds(k * 16, 16), :])
            return 0
        lax.fori_loop(0, ACC_R // 16, zero_k, 0)
        plsc.subcore_barrier()

        def sublist(t2, _):
            plist = 2 * s + (t2 // NSUB)
            sub = t2 % NSUB

            @pl.when(c == 0)
            def _():
                pltpu.sync_copy(codes_hbm.at[rng_s0, plist, sub], sel)

            @pl.when(c == 1)
            def _():
                pltpu.sync_copy(codes_hbm.at[rng_s1, plist, sub], sel)

            cnt = lax.shift_right_logical(jnp.sum(sel[pl.ds(0, 16)]), 4)
            nch = (cnt + (CH - 1)) // CH

            @pl.when(nch > 0)
            def _():
                decode(jnp.int32(0), 0)
                issue_gather(0)

            @pl.when(nch > 1)
            def _():
                decode(jnp.int32(1), 1)
                issue_gather(1)

            def chunk3_loop(jj, _):
                for r in (0, 1, 2):
                    j = 3 * jj + r

                    @pl.when(j < nch)
                    def _(j=j, r=r):
                        wait_gather(r)
                        issue_scatter(r)

                        @pl.when(j + 2 < nch)
                        def _(j=j, r=r):
                            @pl.when(j >= 1)
                            def _():
                                wait_scatter()
                            r2 = (r + 2) % 3
                            decode(j + 2, r2)
                            issue_gather(r2)
                return 0
            lax.fori_loop(0, (MAXCH + 2) // 3, chunk3_loop, 0)

            for t in range(3):
                @pl.when(nch >= t + 1)
                def _():
                    wait_scatter()
            return 0
        lax.fori_loop(0, 2 * NSUB, sublist, 0)
        plsc.subcore_barrier()

        def wb(k, _):
            @pl.when((k % NS == s) & (k * 16 < limit))
            def _():
                pltpu.sync_copy(acc.at[pl.ds(k * 16, 16), :],
                                out_hbm.at[pl.ds(base + k * 16, 16), :])
            return 0
        lax.fori_loop(0, RSZ // 16, wb, 0)
        if rr != NRANGE // NC - 1:
            plsc.subcore_barrier()


def _make_scatter(F):
    return functools.partial(
        pl.kernel,
        out_type=jax.ShapeDtypeStruct((N, F), jnp.float32),
        mesh=plsc.VectorSubcoreMesh(core_axis_name="c", subcore_axis_name="s", num_cores=NC, num_subcores=NS),
        compiler_params=pltpu.CompilerParams(use_tc_tiling_on_sc=False, needs_layout_passes=False),
        scratch_types=[
            pltpu.VMEM((LCAP,), jnp.int32),
            pltpu.VMEM((16, F), jnp.float32),
            pltpu.VMEM((3, CH, F), jnp.float32),
            pltpu.VMEM((8, CH), jnp.int32),
            pltpu.VMEM_SHARED((ACC_R, F), jnp.float32),
            pltpu.SemaphoreType.DMA,
            pltpu.SemaphoreType.DMA,
        ],
    )(functools.partial(_sc_scatter_body, F))


_sc_scatter_h = _make_scatter(H)
_sc_scatter_e = _make_scatter(ENC)


BPAD = 100352
QT = BPAD // 32
C2 = 112
NCH2 = QT // C2


def _sc_sqdist_body(ein_hbm, eout_hbm, emb_hbm, out_hbm,
                    ei, eo, ri, ro, sem):
    c = lax.axis_index("c")
    s = lax.axis_index("s")
    w = c * NS + s
    qbase = w * QT

    pltpu.sync_copy(ein_hbm.at[pl.ds(qbase, QT)], ei)
    pltpu.sync_copy(eout_hbm.at[pl.ds(qbase, QT)], eo)

    pltpu.async_copy(emb_hbm.at[ei.at[pl.ds(0, C2)]], ri.at[0], sem)
    pltpu.async_copy(emb_hbm.at[eo.at[pl.ds(0, C2)]], ro.at[0], sem)

    def chunk(j, _):
        r = jnp.bitwise_and(j, 1)
        pltpu.make_async_copy(emb_hbm.at[pl.ds(0, C2)], ri.at[r], sem).wait()
        pltpu.make_async_copy(emb_hbm.at[pl.ds(0, C2)], ro.at[r], sem).wait()

        @pl.when(j + 1 < NCH2)
        def _():
            pltpu.async_copy(emb_hbm.at[ei.at[pl.ds((j + 1) * C2, C2)]],
                             ri.at[1 - r], sem)
            pltpu.async_copy(emb_hbm.at[eo.at[pl.ds((j + 1) * C2, C2)]],
                             ro.at[1 - r], sem)

        def rowfn(a, _):
            for bcol in range(ENC // 16):
                x = ri[r, a, pl.ds(bcol * 16, 16)]
                y = ro[r, a, pl.ds(bcol * 16, 16)]
                d = x - y
                ri[r, a, pl.ds(bcol * 16, 16)] = d * d
            return 0
        lax.fori_loop(0, C2, rowfn, 0)
        pltpu.sync_copy(ri.at[r], out_hbm.at[pl.ds(qbase + j * C2, C2), :])
        return 0
    lax.fori_loop(0, NCH2, chunk, 0)


_sc_sqdist = functools.partial(
    pl.kernel,
    out_type=jax.ShapeDtypeStruct((BPAD, ENC), jnp.float32),
    mesh=plsc.VectorSubcoreMesh(core_axis_name="c", subcore_axis_name="s", num_cores=NC, num_subcores=NS),
        compiler_params=pltpu.CompilerParams(use_tc_tiling_on_sc=False, needs_layout_passes=False),
    scratch_types=[
        pltpu.VMEM((QT,), jnp.int32),
        pltpu.VMEM((QT,), jnp.int32),
        pltpu.VMEM((2, C2, ENC), jnp.float32),
        pltpu.VMEM((2, C2, ENC), jnp.float32),
        pltpu.SemaphoreType.DMA,
    ],
)(_sc_sqdist_body)


BM = 1000


def _tc_enc1_body(x_ref, w1_ref, d0_ref, d1_ref, xt1_ref, dinv_ref):
    dv = lax.rsqrt(d0_ref[...] + d1_ref[...] + 1.0)
    xw = jnp.dot(x_ref[...], w1_ref[...], preferred_element_type=jnp.float32)
    xt1_ref[...] = xw * dv
    dinv_ref[...] = dv


def _tc_enc1(x, w1, d0, d1):
    return pl.pallas_call(
        _tc_enc1_body,
        grid=(N // BM,),
        in_specs=[
            pl.BlockSpec((BM, D), lambda i: (i, 0)),
            pl.BlockSpec((D, H), lambda i: (0, 0)),
            pl.BlockSpec((BM, 1), lambda i: (i, 0)),
            pl.BlockSpec((BM, 1), lambda i: (i, 0)),
        ],
        out_specs=[
            pl.BlockSpec((BM, H), lambda i: (i, 0)),
            pl.BlockSpec((BM, 1), lambda i: (i, 0)),
        ],
        out_shape=[
            jax.ShapeDtypeStruct((N, H), jnp.float32),
            jax.ShapeDtypeStruct((N, 1), jnp.float32),
        ],
    )(x, w1, d0, d1)


def _tc_enc2_body(s1_ref, xt1_ref, dinv_ref, b1_ref, w2_ref, xt2_ref):
    h = (s1_ref[...] + xt1_ref[...]) * dinv_ref[...] + b1_ref[...]
    h = jnp.maximum(h, 0.0)
    xw = jnp.dot(h, w2_ref[...], preferred_element_type=jnp.float32)
    xt2_ref[...] = xw * dinv_ref[...]


def _tc_enc2(s1, xt1, dinv, b1, w2):
    return pl.pallas_call(
        _tc_enc2_body,
        grid=(N // BM,),
        in_specs=[
            pl.BlockSpec((BM, H), lambda i: (i, 0)),
            pl.BlockSpec((BM, H), lambda i: (i, 0)),
            pl.BlockSpec((BM, 1), lambda i: (i, 0)),
            pl.BlockSpec((1, H), lambda i: (0, 0)),
            pl.BlockSpec((H, ENC), lambda i: (0, 0)),
        ],
        out_specs=pl.BlockSpec((BM, ENC), lambda i: (i, 0)),
        out_shape=jax.ShapeDtypeStruct((N, ENC), jnp.float32),
    )(s1, xt1, dinv, b1, w2)


def _tc_emb_body(s2_ref, xt2_ref, dinv_ref, b2_ref, emb_ref):
    z = (s2_ref[...] + xt2_ref[...]) * dinv_ref[...] + b2_ref[...]
    emb_ref[...] = jnp.maximum(z, 0.0)


def _tc_emb(s2, xt2, dinv, b2):
    return pl.pallas_call(
        _tc_emb_body,
        grid=(N // BM,),
        in_specs=[
            pl.BlockSpec((BM, ENC), lambda i: (i, 0)),
            pl.BlockSpec((BM, ENC), lambda i: (i, 0)),
            pl.BlockSpec((BM, 1), lambda i: (i, 0)),
            pl.BlockSpec((1, ENC), lambda i: (0, 0)),
        ],
        out_specs=pl.BlockSpec((BM, ENC), lambda i: (i, 0)),
        out_shape=jax.ShapeDtypeStruct((N, ENC), jnp.float32),
    )(s2, xt2, dinv, b2)


BD = 2000


def _tc_dec_body(sq_ref, plh_ref, wa_ref, wb_ref, db1_ref, dw2_ref, db2_ref,
                 out_ref):
    z = (jnp.dot(sq_ref[...], wa_ref[...], preferred_element_type=jnp.float32)
         + jnp.dot(plh_ref[...], wb_ref[...],
                   preferred_element_type=jnp.float32)
         + db1_ref[...])
    z = jnp.where(z >= 0.0, z, 0.1 * z)
    z = jnp.dot(z, dw2_ref[...], preferred_element_type=jnp.float32) \
        + db2_ref[...]
    z = jnp.clip(jnp.abs(z), 0.0, 40.0)
    out_ref[...] = 1.0 / (jnp.exp((z - 2.0) * 2.0) + 1.0)


def _tc_dec(sq, plh, wa, wb, db1, dw2, db2):
    return pl.pallas_call(
        _tc_dec_body,
        grid=(B // BD,),
        in_specs=[
            pl.BlockSpec((BD, ENC), lambda i: (i, 0)),
            pl.BlockSpec((BD, PLH), lambda i: (i, 0)),
            pl.BlockSpec((ENC, MLP), lambda i: (0, 0)),
            pl.BlockSpec((PLH, MLP), lambda i: (0, 0)),
            pl.BlockSpec((1, MLP), lambda i: (0, 0)),
            pl.BlockSpec((MLP, 1), lambda i: (0, 0)),
            pl.BlockSpec((1, 1), lambda i: (0, 0)),
        ],
        out_specs=pl.BlockSpec((BD, 1), lambda i: (i, 0)),
        out_shape=jax.ShapeDtypeStruct((B, 1), jnp.float32),
    )(sq, plh, wa, wb, db1, dw2, db2)


def kernel(node_x, gnn_edge_index, edges, plh_x, W1, b1, W2, b2,
           dW1, db1, dW2, db2):
    src = gnn_edge_index[0]
    dst = gnn_edge_index[1]
    srcp = jnp.pad(src, (0, EPAD - E))
    dstp_e = jnp.pad(dst, (0, EPAD - E), constant_values=DST_PAD)

    dstp = jnp.concatenate(
        [dst, jnp.full((NC * NS * DEG_RPT * DEG_W - E,), N, jnp.int32)]
    ).reshape(NC * NS * DEG_RPT, DEG_W)
    degp = _sc_degree(dstp)
    d0 = degp[0, :N].reshape(N, 1)
    d1 = degp[1, :N].reshape(N, 1)

    xt1, dinv = _tc_enc1(node_x, W1, d0, d1)
    codes = _sc_compact(srcp, dstp_e)
    s1 = _sc_scatter_h(codes, xt1)
    xt2 = _tc_enc2(s1, xt1, dinv, b1.reshape(1, H), W2)
    s2 = _sc_scatter_e(codes, xt2)
    emb = _tc_emb(s2, xt2, dinv, b2.reshape(1, ENC))

    ein = jnp.pad(edges[:, 0], (0, BPAD - B))
    eout = jnp.pad(edges[:, 1], (0, BPAD - B))
    sq = _sc_sqdist(ein, eout, emb)

    out = _tc_dec(sq, plh_x, dW1[:ENC], dW1[ENC:], db1.reshape(1, MLP),
                  dW2, db2.reshape(1, 1))
    return out.reshape(-1)
